# baseline (speedup 1.0000x reference)
import jax
import jax.numpy as jnp
from jax import lax
from jax.experimental import pallas as pl
from jax.experimental.pallas import tpu as pltpu
from jax.experimental.pallas import tpu_sc as plsc

_NUM_TASKS = 8
_NUM_MODALITIES = 4
_PRIOR_LEN = 16
_EMBED_DIM = 768
_BATCH = 4096

_ROW = _PRIOR_LEN * _EMBED_DIM
_NROWS = 2 * _BATCH
_NC, _NS = 2, 16
_NW = _NC * _NS
_NG = _NW // 2
_B_PER_G = _BATCH // _NG
_K = 32


def _body(table_hbm, idx_hbm, out_hbm, tbl_v, idx_v, sem):
    wid = lax.axis_index("s") * _NC + lax.axis_index("c")
    half = wid % 2
    base = (wid // 2) * _B_PER_G
    pltpu.sync_copy(idx_hbm.at[wid], idx_v)

    @pl.when(half == 0)
    def _():
        pltpu.sync_copy(table_hbm.at[pl.ds(0, _NUM_TASKS * _ROW)], tbl_v)

    @pl.when(half == 1)
    def _():
        pltpu.sync_copy(
            table_hbm.at[pl.ds(_NUM_TASKS * _ROW, _NUM_MODALITIES * _ROW)],
            tbl_v.at[pl.ds(0, _NUM_MODALITIES * _ROW)])

    def _row_copy(i, r):
        p = (base + i) * 2 + half
        pltpu.async_copy(tbl_v.at[pl.ds(r * _ROW, _ROW // 2)],
                         out_hbm.at[pl.ds(p * _ROW, _ROW // 2)], sem)
        pltpu.async_copy(tbl_v.at[pl.ds(r * _ROW + _ROW // 2, _ROW // 2)],
                         out_hbm.at[pl.ds(p * _ROW + _ROW // 2, _ROW // 2)], sem)

    def _wait_row():
        pltpu.make_async_copy(tbl_v.at[pl.ds(0, _ROW)],
                              out_hbm.at[pl.ds(0, _ROW)], sem).wait()

    for g in range(_K // 16):
        rows0 = idx_v[pl.ds(g * 16, 16)]
        for k in range(16):
            _row_copy(g * 16 + k, rows0[k])

    @pl.loop(_K, _B_PER_G, step=16)
    def _block(i0):
        rows = idx_v[pl.ds(i0, 16)]
        for k in range(16):
            _wait_row()
            _row_copy(i0 + k, rows[k])

    for _ in range(_K):
        _wait_row()


_sc_gather = pl.kernel(
    _body,
    out_type=jax.ShapeDtypeStruct((_NROWS * _ROW,), jnp.float32),
    mesh=plsc.VectorSubcoreMesh(
        core_axis_name="c", subcore_axis_name="s",
        num_cores=_NC, num_subcores=_NS,
    ),
    scratch_types=[
        pltpu.VMEM((_NUM_TASKS * _ROW,), jnp.float32),
        pltpu.VMEM((_B_PER_G,), jnp.int32),
        pltpu.SemaphoreType.DMA,
    ],
)


def kernel(task_table, modality_table, task_idx, modality_idx):
    table = jnp.concatenate(
        [task_table.reshape(_NUM_TASKS * _ROW),
         modality_table.reshape(_NUM_MODALITIES * _ROW)])
    idx = jnp.stack(
        [task_idx.astype(jnp.int32).reshape(_NG, _B_PER_G),
         modality_idx.astype(jnp.int32).reshape(_NG, _B_PER_G)], axis=1)
    idx = idx.reshape(_NW, _B_PER_G)
    out = _sc_gather(table, idx)
    return out.reshape(_BATCH, 2 * _PRIOR_LEN, _EMBED_DIM)
